# baseline (device time: 16763 ns/iter reference)
import jax
import jax.numpy as jnp
from jax import lax
from jax.experimental import pallas as pl
from jax.experimental.pallas import tpu as pltpu

CH = 8


def kernel(x):
    m, n = x.shape
    half = m // 2
    r = half // CH

    def body(x_ref, out_ref, xv, sv, rv,
             in_sems, st_sems, s1_send, s1_recv, s2_send, s2_recv):
        my_x = lax.axis_index("x")
        my_y = lax.axis_index("y")
        my_z = lax.axis_index("z")
        xn = (1 - my_x, my_y, my_z)
        yn = (my_x, 1 - my_y, my_z)
        own = my_x * m
        opp = (1 - my_x) * m

        barrier_sem = pltpu.get_barrier_semaphore()
        for nbr in (xn, yn):
            pl.semaphore_signal(
                barrier_sem, inc=1, device_id=nbr,
                device_id_type=pl.DeviceIdType.MESH,
            )

        s_lo = my_y * half
        k_lo = (1 - my_y) * half
        copies = []
        for c in range(CH):
            rows = pl.ds(s_lo + c * r, r)
            cp = pltpu.make_async_copy(x_ref.at[rows, :], xv.at[rows, :],
                                       in_sems.at[c])
            cp.start()
            copies.append(cp)
        cp_keep = pltpu.make_async_copy(x_ref.at[pl.ds(k_lo, half), :],
                                        xv.at[pl.ds(k_lo, half), :],
                                        in_sems.at[CH])
        cp_keep.start()

        pl.semaphore_wait(barrier_sem, 2)

        sends = []
        for c in range(CH):
            rows = pl.ds(s_lo + c * r, r)
            copies[c].wait()
            sv[rows, :] = xv[rows, :].astype(jnp.bfloat16)
            rd = pltpu.make_async_remote_copy(
                src_ref=sv.at[rows, :],
                dst_ref=rv.at[pl.ds(c * r, r), :],
                send_sem=s1_send.at[c],
                recv_sem=s1_recv.at[c],
                device_id=xn,
                device_id_type=pl.DeviceIdType.MESH,
            )
            rd.start()
            sends.append(rd)

        cp_keep.wait()
        sv[pl.ds(k_lo, half), :] = xv[pl.ds(k_lo, half), :].astype(jnp.bfloat16)
        st_own = pltpu.make_async_copy(sv, out_ref.at[pl.ds(own, m), :],
                                       st_sems.at[CH])
        st_own.start()

        g1 = opp + my_y * half
        for c in range(CH):
            rows_rv = pl.ds(c * r, r)
            rows_g = pl.ds(g1 + c * r, r)
            inbound = pltpu.make_async_remote_copy(
                src_ref=rv.at[rows_rv, :],
                dst_ref=rv.at[rows_rv, :],
                send_sem=s1_send.at[c],
                recv_sem=s1_recv.at[c],
                device_id=xn,
                device_id_type=pl.DeviceIdType.MESH,
            )
            inbound.wait_recv()
            fwd = pltpu.make_async_remote_copy(
                src_ref=rv.at[rows_rv, :],
                dst_ref=out_ref.at[rows_g, :],
                send_sem=s2_send.at[c],
                recv_sem=s2_recv.at[c],
                device_id=yn,
                device_id_type=pl.DeviceIdType.MESH,
            )
            fwd.start()
            sends.append(fwd)
            st = pltpu.make_async_copy(rv.at[rows_rv, :],
                                       out_ref.at[rows_g, :], st_sems.at[c])
            st.start()

        g2 = opp + (1 - my_y) * half
        for c in range(CH):
            rows_g = pl.ds(g2 + c * r, r)
            inbound = pltpu.make_async_remote_copy(
                src_ref=out_ref.at[rows_g, :],
                dst_ref=out_ref.at[rows_g, :],
                send_sem=s2_send.at[c],
                recv_sem=s2_recv.at[c],
                device_id=yn,
                device_id_type=pl.DeviceIdType.MESH,
            )
            inbound.wait_recv()

        st_own.wait()
        for c in range(CH):
            pltpu.make_async_copy(rv.at[pl.ds(c * r, r), :],
                                  out_ref.at[pl.ds(g1 + c * r, r), :],
                                  st_sems.at[c]).wait()
        for rd in sends:
            rd.wait_send()

    return pl.pallas_call(
        body,
        out_shape=jax.ShapeDtypeStruct((2 * m, n), jnp.bfloat16),
        in_specs=[pl.BlockSpec(memory_space=pl.ANY)],
        out_specs=pl.BlockSpec(memory_space=pl.ANY),
        scratch_shapes=[
            pltpu.VMEM((m, n), jnp.float32),
            pltpu.VMEM((m, n), jnp.bfloat16),
            pltpu.VMEM((half, n), jnp.bfloat16),
            pltpu.SemaphoreType.DMA((CH + 1,)),
            pltpu.SemaphoreType.DMA((CH + 1,)),
            pltpu.SemaphoreType.DMA((CH,)),
            pltpu.SemaphoreType.DMA((CH,)),
            pltpu.SemaphoreType.DMA((CH,)),
            pltpu.SemaphoreType.DMA((CH,)),
        ],
        compiler_params=pltpu.CompilerParams(collective_id=0),
    )(x)


# device time: 14712 ns/iter; 1.1394x vs baseline; 1.1394x over previous
import jax
import jax.numpy as jnp
from jax import lax
from jax.experimental import pallas as pl
from jax.experimental.pallas import tpu as pltpu

R = 80
DELTA = 112
NF = (512 - DELTA) // R


def kernel(x):
    m, n = x.shape
    half = m // 2
    front = half - DELTA

    def body(x_ref, out_ref, s1_send, s1_recv, s2_send, s2_recv):
        my_x = lax.axis_index("x")
        my_y = lax.axis_index("y")
        my_z = lax.axis_index("z")
        xn = (1 - my_x, my_y, my_z)
        yn = (my_x, 1 - my_y, my_z)
        own = my_x * m
        opp = (1 - my_x) * m

        barrier_sem = pltpu.get_barrier_semaphore()
        for nbr in (xn, yn):
            pl.semaphore_signal(
                barrier_sem, inc=1, device_id=nbr,
                device_id_type=pl.DeviceIdType.MESH,
            )

        s_lo = my_y * half
        k_lo = (1 - my_y) * half
        out_ref[pl.ds(own + s_lo, half), :] = (
            x_ref[pl.ds(s_lo, half), :].astype(jnp.bfloat16))
        pl.semaphore_wait(barrier_sem, 2)

        def send_x(rows_g, sem_slot):
            rd = pltpu.make_async_remote_copy(
                src_ref=out_ref.at[rows_g, :], dst_ref=out_ref.at[rows_g, :],
                send_sem=s1_send.at[sem_slot], recv_sem=s1_recv.at[sem_slot],
                device_id=xn, device_id_type=pl.DeviceIdType.MESH,
            )
            rd.start()
            return rd

        sends = []
        for c in range(NF):
            sends.append(send_x(pl.ds(own + s_lo + c * R, R), c))
        sends.append(send_x(pl.ds(own + s_lo + front, DELTA), NF))
        sends.append(send_x(pl.ds(own + k_lo + front, DELTA), NF + 1))

        out_ref[pl.ds(own + k_lo, half), :] = (
            x_ref[pl.ds(k_lo, half), :].astype(jnp.bfloat16))

        g1 = opp + my_y * half
        for c in range(NF):
            rows = pl.ds(g1 + c * R, R)
            inbound = pltpu.make_async_remote_copy(
                src_ref=out_ref.at[rows, :], dst_ref=out_ref.at[rows, :],
                send_sem=s1_send.at[c], recv_sem=s1_recv.at[c],
                device_id=xn, device_id_type=pl.DeviceIdType.MESH,
            )
            inbound.wait_recv()
            fwd = pltpu.make_async_remote_copy(
                src_ref=out_ref.at[rows, :], dst_ref=out_ref.at[rows, :],
                send_sem=s2_send.at[c], recv_sem=s2_recv.at[c],
                device_id=yn, device_id_type=pl.DeviceIdType.MESH,
            )
            fwd.start()
            sends.append(fwd)

        g2 = opp + (1 - my_y) * half
        tails = ((pl.ds(g1 + front, DELTA), s1_recv, NF, xn),
                 (pl.ds(g2 + front, DELTA), s1_recv, NF + 1, xn))
        for rows, sem, slot, dev in tails:
            pltpu.make_async_remote_copy(
                src_ref=out_ref.at[rows, :], dst_ref=out_ref.at[rows, :],
                send_sem=s1_send.at[slot], recv_sem=sem.at[slot],
                device_id=dev, device_id_type=pl.DeviceIdType.MESH,
            ).wait_recv()
        for c in range(NF):
            rows = pl.ds(g2 + c * R, R)
            pltpu.make_async_remote_copy(
                src_ref=out_ref.at[rows, :], dst_ref=out_ref.at[rows, :],
                send_sem=s2_send.at[c], recv_sem=s2_recv.at[c],
                device_id=yn, device_id_type=pl.DeviceIdType.MESH,
            ).wait_recv()

        for rd in sends:
            rd.wait_send()

    return pl.pallas_call(
        body,
        out_shape=jax.ShapeDtypeStruct((2 * m, n), jnp.bfloat16),
        in_specs=[pl.BlockSpec(memory_space=pltpu.VMEM)],
        out_specs=pl.BlockSpec(memory_space=pltpu.VMEM),
        scratch_shapes=[
            pltpu.SemaphoreType.DMA((NF + 2,)),
            pltpu.SemaphoreType.DMA((NF + 2,)),
            pltpu.SemaphoreType.DMA((NF,)),
            pltpu.SemaphoreType.DMA((NF,)),
        ],
        compiler_params=pltpu.CompilerParams(collective_id=0),
    )(x)
